# baseline (speedup 1.0000x reference)
import functools
import jax
import jax.numpy as jnp
from jax import lax
from jax.experimental import pallas as pl
from jax.experimental.pallas import tpu as pltpu
from jax.experimental.pallas import tpu_sc as plsc


def _sc_body(x_hbm, out_hbm, buf_in, buf_dil, buf_zero):
    B, H, W, C = x_hbm.shape
    NC, NS = 2, 16
    rows_per_w = (B * H) // (NC * NS)
    wid = lax.axis_index("s") * NC + lax.axis_index("c")
    zv = jnp.zeros((16,), jnp.float32)
    nvec = C // 16

    Wc = W // 2

    def _zero_row(r, _):
        for j in range(nvec):
            buf_dil[r, pl.ds(16 * j, 16)] = zv
            buf_zero[r, pl.ds(16 * j, 16)] = zv
        return 0

    lax.fori_loop(0, 2 * Wc, _zero_row, 0)

    def _do_row(k, _):
        row = wid * rows_per_w + k
        b = row // H
        h = row % H
        for s in range(2):
            pltpu.sync_copy(x_hbm.at[b, h, pl.ds(s * Wc, Wc)], buf_in)

            def _dilate(w, _):
                for j in range(nvec):
                    buf_dil[2 * w, pl.ds(16 * j, 16)] = (
                        buf_in[w, pl.ds(16 * j, 16)])
                return 0

            lax.fori_loop(0, Wc, _dilate, 0)
            pltpu.sync_copy(buf_dil,
                            out_hbm.at[b, 2 * h, pl.ds(s * 2 * Wc, 2 * Wc)])
            pltpu.sync_copy(buf_zero,
                            out_hbm.at[b, 2 * h + 1,
                                       pl.ds(s * 2 * Wc, 2 * Wc)])
        return 0

    lax.fori_loop(0, rows_per_w, _do_row, 0)


def kernel(inputs):
    B, H, W, C = inputs.shape
    mesh = plsc.VectorSubcoreMesh(core_axis_name="c", subcore_axis_name="s")
    k = functools.partial(
        pl.kernel,
        mesh=mesh,
        out_type=jax.ShapeDtypeStruct((B, 2 * H, 2 * W, C), inputs.dtype),
        scratch_types=[
            pltpu.VMEM((W // 2, C), jnp.float32),
            pltpu.VMEM((W, C), jnp.float32),
            pltpu.VMEM((W, C), jnp.float32),
        ],
    )(_sc_body)
    return k(inputs)


# speedup vs baseline: 1.2232x; 1.2232x over previous
import functools
import jax
import jax.numpy as jnp
from jax import lax
from jax.experimental import pallas as pl
from jax.experimental.pallas import tpu as pltpu
from jax.experimental.pallas import tpu_sc as plsc


def _sc_body(x_hbm, out_hbm, in0, in1, dil0, dil1, buf_zero,
             gsem0, gsem1, dsem0, dsem1, zsem):
    B, H, W, C = x_hbm.shape
    NC, NS = 2, 16
    Wc = W // 2
    n_half = 2 * B * H // (NC * NS)
    wid = lax.axis_index("s") * NC + lax.axis_index("c")
    zv = jnp.zeros((16,), jnp.float32)
    nvec = C // 16

    def _zero_row(r, _):
        for j in range(nvec):
            dil0[r, pl.ds(16 * j, 16)] = zv
            dil1[r, pl.ds(16 * j, 16)] = zv
            buf_zero[r, pl.ds(16 * j, 16)] = zv
        return 0

    lax.fori_loop(0, 2 * Wc, _zero_row, 0)

    def _src(t):
        g = wid * n_half + t
        b = g // (2 * H)
        r = g % (2 * H)
        return b, r // 2, r % 2

    def _gather(t, buf, sem):
        b, h, s = _src(t)
        return pltpu.async_copy(x_hbm.at[b, h, pl.ds(s * Wc, Wc)], buf, sem)

    def _wait_gather(t, buf, sem):
        b, h, s = _src(t)
        pltpu.make_async_copy(
            x_hbm.at[b, h, pl.ds(s * Wc, Wc)], buf, sem).wait()

    def _scatter(t, buf, sem, odd):
        b, h, s = _src(t)
        dst = out_hbm.at[b, 2 * h + odd, pl.ds(s * 2 * Wc, 2 * Wc)]
        return pltpu.async_copy(buf, dst, sem)

    def _wait_scatter(t, buf, sem, odd):
        b, h, s = _src(t)
        dst = out_hbm.at[b, 2 * h + odd, pl.ds(s * 2 * Wc, 2 * Wc)]
        pltpu.make_async_copy(buf, dst, sem).wait()

    def _dilate(buf_in, buf_dil):
        def body(w, _):
            for j in range(nvec):
                buf_dil[2 * w, pl.ds(16 * j, 16)] = (
                    buf_in[w, pl.ds(16 * j, 16)])
            return 0
        lax.fori_loop(0, Wc, body, 0)

    _gather(0, in0, gsem0)

    def _iter(i, _):
        tA = 2 * i
        tB = 2 * i + 1

        @pl.when(i > 0)
        def _():
            _wait_scatter(tA - 2, dil0, dsem0, 0)
        _wait_gather(tA, in0, gsem0)
        _gather(tB, in1, gsem1)
        _dilate(in0, dil0)
        _scatter(tA, dil0, dsem0, 0)

        @pl.when(i > 0)
        def _():
            _wait_scatter(tA - 1, buf_zero, zsem, 1)
        _scatter(tA, buf_zero, zsem, 1)

        @pl.when(i > 0)
        def _():
            _wait_scatter(tB - 2, dil1, dsem1, 0)
        _wait_gather(tB, in1, gsem1)

        @pl.when(i < (n_half // 2) - 1)
        def _():
            _gather(tB + 1, in0, gsem0)
        _dilate(in1, dil1)
        _scatter(tB, dil1, dsem1, 0)
        _wait_scatter(tB - 1, buf_zero, zsem, 1)
        _scatter(tB, buf_zero, zsem, 1)
        return 0

    lax.fori_loop(0, n_half // 2, _iter, 0)

    _wait_scatter(n_half - 2, dil0, dsem0, 0)
    _wait_scatter(n_half - 1, dil1, dsem1, 0)
    _wait_scatter(n_half - 1, buf_zero, zsem, 1)


def kernel(inputs):
    B, H, W, C = inputs.shape
    mesh = plsc.VectorSubcoreMesh(core_axis_name="c", subcore_axis_name="s")
    k = functools.partial(
        pl.kernel,
        mesh=mesh,
        out_type=jax.ShapeDtypeStruct((B, 2 * H, 2 * W, C), inputs.dtype),
        scratch_types=[
            pltpu.VMEM((W // 2, C), jnp.float32),
            pltpu.VMEM((W // 2, C), jnp.float32),
            pltpu.VMEM((W, C), jnp.float32),
            pltpu.VMEM((W, C), jnp.float32),
            pltpu.VMEM((W, C), jnp.float32),
            pltpu.SemaphoreType.DMA,
            pltpu.SemaphoreType.DMA,
            pltpu.SemaphoreType.DMA,
            pltpu.SemaphoreType.DMA,
            pltpu.SemaphoreType.DMA,
        ],
    )(_sc_body)
    return k(inputs)


# speedup vs baseline: 1.2272x; 1.0033x over previous
import functools
import jax
import jax.numpy as jnp
from jax import lax
from jax.experimental import pallas as pl
from jax.experimental.pallas import tpu as pltpu
from jax.experimental.pallas import tpu_sc as plsc


def _sc_body(x_hbm, out_hbm, in0, in1, dil0, dil1, buf_zero, shared_zero,
             gsem0, gsem1, dsem0, dsem1, zsem):
    B, H, W, C = x_hbm.shape
    NC, NS = 2, 16
    Wc = W // 2
    n_half = 2 * B * H // (NC * NS)
    wid = lax.axis_index("s") * NC + lax.axis_index("c")
    zv = jnp.zeros((16,), jnp.float32)
    nvec = C // 16

    def _zero_row(r, _):
        for j in range(nvec):
            dil0[r, pl.ds(16 * j, 16)] = zv
            dil1[r, pl.ds(16 * j, 16)] = zv
            buf_zero[r, pl.ds(16 * j, 16)] = zv
        return 0

    lax.fori_loop(0, 2 * Wc, _zero_row, 0)

    @pl.when(lax.axis_index("s") == 0)
    def _():
        pltpu.sync_copy(buf_zero, shared_zero.at[pl.ds(0, 2 * Wc)])
        pltpu.sync_copy(buf_zero, shared_zero.at[pl.ds(2 * Wc, 2 * Wc)])

    plsc.subcore_barrier()

    def _src(t):
        g = wid * n_half + t
        b = g // (2 * H)
        r = g % (2 * H)
        return b, r // 2, r % 2

    def _gather(t, buf, sem):
        b, h, s = _src(t)
        return pltpu.async_copy(x_hbm.at[b, h, pl.ds(s * Wc, Wc)], buf, sem)

    def _wait_gather(t, buf, sem):
        b, h, s = _src(t)
        pltpu.make_async_copy(
            x_hbm.at[b, h, pl.ds(s * Wc, Wc)], buf, sem).wait()

    def _scatter(t, buf, sem, odd):
        b, h, s = _src(t)
        dst = out_hbm.at[b, 2 * h + odd, pl.ds(s * 2 * Wc, 2 * Wc)]
        return pltpu.async_copy(buf, dst, sem)

    def _wait_scatter(t, buf, sem, odd):
        b, h, s = _src(t)
        dst = out_hbm.at[b, 2 * h + odd, pl.ds(s * 2 * Wc, 2 * Wc)]
        pltpu.make_async_copy(buf, dst, sem).wait()

    def _scatter_zero_row(t):
        b, h, _ = _src(t)
        return pltpu.async_copy(shared_zero, out_hbm.at[b, 2 * h + 1], zsem)

    def _wait_zero_row(t):
        b, h, _ = _src(t)
        pltpu.make_async_copy(
            shared_zero, out_hbm.at[b, 2 * h + 1], zsem).wait()

    def _dilate(buf_in, buf_dil):
        def body(w, _):
            for j in range(nvec):
                buf_dil[2 * w, pl.ds(16 * j, 16)] = (
                    buf_in[w, pl.ds(16 * j, 16)])
            return 0
        lax.fori_loop(0, Wc, body, 0)

    _gather(0, in0, gsem0)

    def _iter(i, _):
        tA = 2 * i
        tB = 2 * i + 1

        @pl.when(i > 0)
        def _():
            _wait_scatter(tA - 2, dil0, dsem0, 0)
        _wait_gather(tA, in0, gsem0)
        _gather(tB, in1, gsem1)
        _dilate(in0, dil0)
        _scatter(tA, dil0, dsem0, 0)

        @pl.when(i > 0)
        def _():
            _wait_zero_row(tA - 2)
        _scatter_zero_row(tA)

        @pl.when(i > 0)
        def _():
            _wait_scatter(tB - 2, dil1, dsem1, 0)
        _wait_gather(tB, in1, gsem1)

        @pl.when(i < (n_half // 2) - 1)
        def _():
            _gather(tB + 1, in0, gsem0)
        _dilate(in1, dil1)
        _scatter(tB, dil1, dsem1, 0)
        return 0

    lax.fori_loop(0, n_half // 2, _iter, 0)

    _wait_scatter(n_half - 2, dil0, dsem0, 0)
    _wait_scatter(n_half - 1, dil1, dsem1, 0)
    _wait_zero_row(n_half - 2)


def kernel(inputs):
    B, H, W, C = inputs.shape
    mesh = plsc.VectorSubcoreMesh(core_axis_name="c", subcore_axis_name="s")
    k = functools.partial(
        pl.kernel,
        mesh=mesh,
        out_type=jax.ShapeDtypeStruct((B, 2 * H, 2 * W, C), inputs.dtype),
        scratch_types=[
            pltpu.VMEM((W // 2, C), jnp.float32),
            pltpu.VMEM((W // 2, C), jnp.float32),
            pltpu.VMEM((W, C), jnp.float32),
            pltpu.VMEM((W, C), jnp.float32),
            pltpu.VMEM((W, C), jnp.float32),
            pltpu.VMEM_SHARED((2 * W, C), jnp.float32),
            pltpu.SemaphoreType.DMA,
            pltpu.SemaphoreType.DMA,
            pltpu.SemaphoreType.DMA,
            pltpu.SemaphoreType.DMA,
            pltpu.SemaphoreType.DMA,
        ],
    )(_sc_body)
    return k(inputs)
